# baseline (speedup 1.0000x reference)
import functools

import jax
import jax.numpy as jnp
from jax import lax
from jax.experimental import pallas as pl
from jax.experimental.pallas import tpu as pltpu
from jax.experimental.pallas import tpu_sc as plsc

N_NODES = 10000
D_FEAT = 128
N_EDGES = 320000
HIDDEN = 16

NUM_WORKERS = 32
CHUNK = N_EDGES // NUM_WORKERS
LANES = 16
UNROLL = 12

TAB_N = 10240
NODES_PER_BLK = 5120
GRID = TAB_N // NODES_PER_BLK


def _node_scalars_body(x_ref, wnb_ref, bnb_ref, wself_ref, bself_ref,
                       watt_ref, batt_ref, snb_ref, sself_ref):
    xv = x_ref[...]
    h1 = jnp.maximum(
        lax.dot_general(xv, wnb_ref[...], (((1,), (1,)), ((), ())),
                        preferred_element_type=jnp.float32)
        + bnb_ref[...], 0.0)
    h2 = jnp.maximum(
        lax.dot_general(xv, wself_ref[...], (((1,), (1,)), ((), ())),
                        preferred_element_type=jnp.float32)
        + bself_ref[...], 0.0)
    wa = watt_ref[...]
    wa1 = wa[0:HIDDEN].reshape(1, HIDDEN)
    wa2 = wa[HIDDEN:].reshape(1, HIDDEN)

    def rows(h, wa_row):
        s = lax.dot_general(wa_row, h, (((1,), (1,)), ((), ())),
                            preferred_element_type=jnp.float32)
        return s.reshape(NODES_PER_BLK)

    snb_ref[...] = rows(h1, wa1) + batt_ref[0]
    sself_ref[...] = rows(h2, wa2)


def _node_scalars(x, W_nb, b_nb, W_self, b_self, W_att, b_att):
    return pl.pallas_call(
        _node_scalars_body,
        grid=(GRID,),
        in_specs=[
            pl.BlockSpec((NODES_PER_BLK, D_FEAT), lambda i: (i, 0)),
            pl.BlockSpec((HIDDEN, D_FEAT), lambda i: (0, 0)),
            pl.BlockSpec((HIDDEN,), lambda i: (0,)),
            pl.BlockSpec((HIDDEN, D_FEAT), lambda i: (0, 0)),
            pl.BlockSpec((HIDDEN,), lambda i: (0,)),
            pl.BlockSpec((2 * HIDDEN,), lambda i: (0,)),
            pl.BlockSpec((1,), lambda i: (0,)),
        ],
        out_specs=[
            pl.BlockSpec((NODES_PER_BLK,), lambda i: (i,)),
            pl.BlockSpec((NODES_PER_BLK,), lambda i: (i,)),
        ],
        out_shape=[
            jax.ShapeDtypeStruct((TAB_N,), jnp.float32),
            jax.ShapeDtypeStruct((TAB_N,), jnp.float32),
        ],
    )(x, W_nb.T, b_nb, W_self.T, b_self, W_att.reshape(2 * HIDDEN), b_att)


ETILE = 128
N_ETILES = N_EDGES // ETILE
TPW = N_ETILES // NUM_WORKERS
MAIN = TPW * ETILE
TAIL_T0 = TPW * NUM_WORKERS
N_TAIL = N_ETILES - TAIL_T0


HALF = MAIN // 2


def _edge_gate_body(snb_hbm, sself_hbm, edge_hbm,
                    mask_hbm, psum_hbm,
                    snb_v, sself_v, e0_v, e1_v, et_v, mask_v, mt_v, acc_v,
                    sem_t, sem_a, sem_b, sem_c):
    wid = lax.axis_index("s") * 2 + lax.axis_index("c")
    base = wid * MAIN
    c1 = pltpu.async_copy(snb_hbm, snb_v, sem_t)
    c2 = pltpu.async_copy(sself_hbm, sself_v, sem_t)
    c3a = pltpu.async_copy(edge_hbm.at[:, pl.ds(base, HALF)], e0_v, sem_a)
    c3b = pltpu.async_copy(edge_hbm.at[:, pl.ds(base + HALF, HALF)], e1_v,
                           sem_b)
    c4 = pltpu.async_copy(
        edge_hbm.at[:, pl.ds((TAIL_T0 + wid % N_TAIL) * ETILE, ETILE)],
        et_v, sem_c)
    c1.wait()
    c2.wait()
    c3a.wait()

    def gate(idx_r, idx_c):
        s1 = plsc.load_gather(snb_v, [idx_r])
        s2 = plsc.load_gather(sself_v, [idx_c])
        la = s1 + s2
        return jnp.minimum(1.01 / (1.0 + jnp.exp(-la)), 1.0)

    @plsc.parallel_loop(0, HALF, LANES, unroll=UNROLL,
                        carry=jnp.zeros((LANES,), jnp.float32))
    def acc0(off, acc_in):
        m = gate(e0_v[0, pl.ds(off, LANES)], e0_v[1, pl.ds(off, LANES)])
        mask_v[pl.ds(off, LANES)] = m
        return acc_in + m

    c3b.wait()

    @plsc.parallel_loop(0, HALF, LANES, unroll=UNROLL, carry=acc0)
    def acc(off, acc_in):
        m = gate(e1_v[0, pl.ds(off, LANES)], e1_v[1, pl.ds(off, LANES)])
        mask_v[pl.ds(HALF + off, LANES)] = m
        return acc_in + m

    pltpu.sync_copy(mask_v, mask_hbm.at[pl.ds(base, MAIN)])
    c4.wait()

    @pl.when(wid < N_TAIL)
    def _tail():
        @plsc.parallel_loop(0, ETILE, LANES, unroll=ETILE // LANES,
                            carry=acc)
        def acc2(off, acc_in):
            m = gate(et_v[0, pl.ds(off, LANES)], et_v[1, pl.ds(off, LANES)])
            mt_v[pl.ds(off, LANES)] = m
            return acc_in + m

        acc_v[...] = acc2
        pltpu.sync_copy(
            mt_v, mask_hbm.at[pl.ds((TAIL_T0 + wid) * ETILE, ETILE)])

    @pl.when(wid >= N_TAIL)
    def _no_tail():
        acc_v[...] = acc

    pltpu.sync_copy(acc_v, psum_hbm.at[wid])


def _edge_gate(s_nb, s_self, edge_index):
    mesh = plsc.VectorSubcoreMesh(core_axis_name="c", subcore_axis_name="s")
    fn = functools.partial(
        pl.kernel,
        mesh=mesh,
        compiler_params=pltpu.CompilerParams(needs_layout_passes=False),
        out_type=[
            jax.ShapeDtypeStruct((N_EDGES,), jnp.float32),
            jax.ShapeDtypeStruct((NUM_WORKERS, LANES), jnp.float32),
        ],
        scratch_types=[
            pltpu.VMEM((TAB_N,), jnp.float32),
            pltpu.VMEM((TAB_N,), jnp.float32),
            pltpu.VMEM((2, HALF), jnp.int32),
            pltpu.VMEM((2, HALF), jnp.int32),
            pltpu.VMEM((2, ETILE), jnp.int32),
            pltpu.VMEM((MAIN,), jnp.float32),
            pltpu.VMEM((ETILE,), jnp.float32),
            pltpu.VMEM((LANES,), jnp.float32),
            pltpu.SemaphoreType.DMA,
            pltpu.SemaphoreType.DMA,
            pltpu.SemaphoreType.DMA,
            pltpu.SemaphoreType.DMA,
        ],
    )(_edge_gate_body)
    return fn(s_nb, s_self, edge_index)


def _sum_body(p_ref, out_ref):
    out_ref[...] = jnp.sum(p_ref[...]).reshape(1, 1)


def _sum_partials(partials):
    return pl.pallas_call(
        _sum_body,
        out_shape=jax.ShapeDtypeStruct((1, 1), jnp.float32),
    )(partials)


def kernel(x, edge_index, W_nb, b_nb, W_self, b_self, W_att, b_att):
    s_nb, s_self = _node_scalars(x, W_nb, b_nb, W_self, b_self, W_att, b_att)
    mask_flat, partials = _edge_gate(s_nb, s_self,
                                     edge_index.astype(jnp.int32))
    mask_sum = _sum_partials(partials).reshape(())
    return mask_flat.reshape(N_EDGES, 1), mask_sum


# speedup vs baseline: 1.1965x; 1.1965x over previous
import functools

import jax
import jax.numpy as jnp
from jax import lax
from jax.experimental import pallas as pl
from jax.experimental.pallas import tpu as pltpu
from jax.experimental.pallas import tpu_sc as plsc

N_NODES = 10000
D_FEAT = 128
N_EDGES = 320000
HIDDEN = 16

NUM_WORKERS = 32
CHUNK = N_EDGES // NUM_WORKERS
LANES = 16
UNROLL = 12

TAB_N = 10240
NODES_PER_BLK = 5120
GRID = TAB_N // NODES_PER_BLK


def _node_scalars_body(x_ref, wnb_ref, bnb_ref, wself_ref, bself_ref,
                       watt_ref, batt_ref, snb_ref, sself_ref):
    xv = x_ref[...]
    h1 = jnp.maximum(
        lax.dot_general(xv, wnb_ref[...], (((1,), (1,)), ((), ())),
                        preferred_element_type=jnp.float32)
        + bnb_ref[...], 0.0)
    h2 = jnp.maximum(
        lax.dot_general(xv, wself_ref[...], (((1,), (1,)), ((), ())),
                        preferred_element_type=jnp.float32)
        + bself_ref[...], 0.0)
    wa = watt_ref[...]
    wa1 = wa[0:HIDDEN].reshape(1, HIDDEN)
    wa2 = wa[HIDDEN:].reshape(1, HIDDEN)

    def rows(h, wa_row):
        s = lax.dot_general(wa_row, h, (((1,), (1,)), ((), ())),
                            preferred_element_type=jnp.float32)
        return s.reshape(NODES_PER_BLK)

    snb_ref[...] = rows(h1, wa1) + batt_ref[0]
    sself_ref[...] = rows(h2, wa2)


def _node_scalars(x, W_nb, b_nb, W_self, b_self, W_att, b_att):
    return pl.pallas_call(
        _node_scalars_body,
        grid=(GRID,),
        in_specs=[
            pl.BlockSpec((NODES_PER_BLK, D_FEAT), lambda i: (i, 0)),
            pl.BlockSpec((HIDDEN, D_FEAT), lambda i: (0, 0)),
            pl.BlockSpec((HIDDEN,), lambda i: (0,)),
            pl.BlockSpec((HIDDEN, D_FEAT), lambda i: (0, 0)),
            pl.BlockSpec((HIDDEN,), lambda i: (0,)),
            pl.BlockSpec((2 * HIDDEN,), lambda i: (0,)),
            pl.BlockSpec((1,), lambda i: (0,)),
        ],
        out_specs=[
            pl.BlockSpec((NODES_PER_BLK,), lambda i: (i,)),
            pl.BlockSpec((NODES_PER_BLK,), lambda i: (i,)),
        ],
        out_shape=[
            jax.ShapeDtypeStruct((TAB_N,), jnp.float32),
            jax.ShapeDtypeStruct((TAB_N,), jnp.float32),
        ],
    )(x, W_nb.T, b_nb, W_self.T, b_self, W_att.reshape(2 * HIDDEN), b_att)


ETILE = 128
N_ETILES = N_EDGES // ETILE
TPW = N_ETILES // NUM_WORKERS
MAIN = TPW * ETILE
TAIL_T0 = TPW * NUM_WORKERS
N_TAIL = N_ETILES - TAIL_T0


HALF = MAIN // 2


def _edge_gate_body(snb_hbm, sself_hbm, edge_hbm,
                    mask_hbm, psum_hbm,
                    snb_v, sself_v, e0_v, e1_v, et_v, mask_v, mt_v, acc_v,
                    sem_t, sem_a, sem_b, sem_c):
    wid = lax.axis_index("s") * 2 + lax.axis_index("c")
    base = wid * MAIN
    c1 = pltpu.async_copy(snb_hbm, snb_v, sem_t)
    c2 = pltpu.async_copy(sself_hbm, sself_v, sem_t)
    c3a = pltpu.async_copy(edge_hbm.at[:, pl.ds(base, HALF)], e0_v, sem_a)
    c3b = pltpu.async_copy(edge_hbm.at[:, pl.ds(base + HALF, HALF)], e1_v,
                           sem_b)
    c4 = pltpu.async_copy(
        edge_hbm.at[:, pl.ds((TAIL_T0 + wid % N_TAIL) * ETILE, ETILE)],
        et_v, sem_c)
    c1.wait()
    c2.wait()
    c3a.wait()

    def gate(idx_r, idx_c):
        s1 = plsc.load_gather(snb_v, [idx_r])
        s2 = plsc.load_gather(sself_v, [idx_c])
        la = s1 + s2
        return jnp.minimum(1.01 / (1.0 + jnp.exp(-la)), 1.0)

    @plsc.parallel_loop(0, HALF, LANES, unroll=UNROLL,
                        carry=jnp.zeros((LANES,), jnp.float32))
    def acc0(off, acc_in):
        m = gate(e0_v[0, pl.ds(off, LANES)], e0_v[1, pl.ds(off, LANES)])
        mask_v[pl.ds(off, LANES)] = m
        return acc_in + m

    c3b.wait()

    @plsc.parallel_loop(0, HALF, LANES, unroll=UNROLL, carry=acc0)
    def acc(off, acc_in):
        m = gate(e1_v[0, pl.ds(off, LANES)], e1_v[1, pl.ds(off, LANES)])
        mask_v[pl.ds(HALF + off, LANES)] = m
        return acc_in + m

    pltpu.sync_copy(mask_v, mask_hbm.at[0, pl.ds(base, MAIN)])
    c4.wait()

    @pl.when(wid < N_TAIL)
    def _tail():
        @plsc.parallel_loop(0, ETILE, LANES, unroll=ETILE // LANES,
                            carry=acc)
        def acc2(off, acc_in):
            m = gate(et_v[0, pl.ds(off, LANES)], et_v[1, pl.ds(off, LANES)])
            mt_v[pl.ds(off, LANES)] = m
            return acc_in + m

        acc_v[...] = acc2
        pltpu.sync_copy(
            mt_v, mask_hbm.at[0, pl.ds((TAIL_T0 + wid) * ETILE, ETILE)])

    @pl.when(wid >= N_TAIL)
    def _no_tail():
        acc_v[...] = acc

    pltpu.sync_copy(acc_v, psum_hbm.at[wid])


def _edge_gate(s_nb, s_self, edge_index):
    mesh = plsc.VectorSubcoreMesh(core_axis_name="c", subcore_axis_name="s")
    fn = functools.partial(
        pl.kernel,
        mesh=mesh,
        compiler_params=pltpu.CompilerParams(needs_layout_passes=False),
        out_type=[
            jax.ShapeDtypeStruct((1, N_EDGES), jnp.float32),
            jax.ShapeDtypeStruct((NUM_WORKERS, LANES), jnp.float32),
        ],
        scratch_types=[
            pltpu.VMEM((TAB_N,), jnp.float32),
            pltpu.VMEM((TAB_N,), jnp.float32),
            pltpu.VMEM((2, HALF), jnp.int32),
            pltpu.VMEM((2, HALF), jnp.int32),
            pltpu.VMEM((2, ETILE), jnp.int32),
            pltpu.VMEM((MAIN,), jnp.float32),
            pltpu.VMEM((ETILE,), jnp.float32),
            pltpu.VMEM((LANES,), jnp.float32),
            pltpu.SemaphoreType.DMA,
            pltpu.SemaphoreType.DMA,
            pltpu.SemaphoreType.DMA,
            pltpu.SemaphoreType.DMA,
        ],
    )(_edge_gate_body)
    return fn(s_nb, s_self, edge_index)


def _sum_body(p_ref, out_ref):
    out_ref[...] = jnp.sum(p_ref[...]).reshape(1, 1)


def _sum_partials(partials):
    return pl.pallas_call(
        _sum_body,
        out_shape=jax.ShapeDtypeStruct((1, 1), jnp.float32),
    )(partials)


def kernel(x, edge_index, W_nb, b_nb, W_self, b_self, W_att, b_att):
    s_nb, s_self = _node_scalars(x, W_nb, b_nb, W_self, b_self, W_att, b_att)
    mask_flat, partials = _edge_gate(s_nb, s_self,
                                     edge_index.astype(jnp.int32))
    mask_sum = _sum_partials(partials).reshape(())
    return mask_flat.reshape(N_EDGES, 1), mask_sum
